# baseline (speedup 1.0000x reference)
import functools

import jax
import jax.numpy as jnp
from jax import lax
from jax.experimental import pallas as pl
from jax.experimental.pallas import tpu as pltpu
from jax.experimental.pallas import tpu_sc as plsc

_NUM_CORES = 2
_NUM_SUBCORES = 16
_NUM_WORKERS = _NUM_CORES * _NUM_SUBCORES
_LANES = 16


@functools.lru_cache(maxsize=None)
def _make_gather_pool(B, CTX, D):
    bpw = B // _NUM_WORKERS
    ipw = bpw * CTX
    mesh = plsc.VectorSubcoreMesh(core_axis_name="c", subcore_axis_name="s")

    @functools.partial(
        pl.kernel,
        mesh=mesh,
        out_type=jax.ShapeDtypeStruct((B, D), jnp.float32),
        scratch_types=[
            pltpu.VMEM((ipw,), jnp.int32),
            pltpu.VMEM((ipw, D), jnp.float32),
            pltpu.VMEM((bpw, D), jnp.float32),
            pltpu.SemaphoreType.DMA,
        ],
        compiler_params=pltpu.CompilerParams(use_tc_tiling_on_sc=False),
    )
    def gather_pool(ids_hbm, table_hbm, out_hbm, idx_v, rows_v, pooled_v, sem):
        wid = lax.axis_index("s") * _NUM_CORES + lax.axis_index("c")
        base = wid * ipw
        pltpu.sync_copy(ids_hbm.at[pl.ds(base, ipw)], idx_v)
        pltpu.async_copy(table_hbm.at[idx_v], rows_v, sem).wait()
        scale = jnp.float32(1.0 / CTX)

        def body(b, carry):
            for d in range(D // _LANES):
                sl = pl.ds(d * _LANES, _LANES)
                acc = rows_v[b * CTX, sl]
                for c in range(1, CTX):
                    acc = acc + rows_v[b * CTX + c, sl]
                pooled_v[b, sl] = acc * scale
            return carry

        lax.fori_loop(0, bpw, body, 0)
        pltpu.sync_copy(pooled_v, out_hbm.at[pl.ds(wid * bpw, bpw)])

    return gather_pool


@functools.lru_cache(maxsize=None)
def _make_project(B, D, V, TV):
    def body(x_ref, w_ref, o_ref):
        o_ref[...] = lax.dot_general(
            x_ref[...], w_ref[...],
            dimension_numbers=(((1,), (1,)), ((), ())),
            preferred_element_type=jnp.float32,
        )

    return pl.pallas_call(
        body,
        grid=(pl.cdiv(V, TV),),
        in_specs=[
            pl.BlockSpec((B, D), lambda i: (0, 0)),
            pl.BlockSpec((TV, D), lambda i: (i, 0)),
        ],
        out_specs=pl.BlockSpec((B, TV), lambda i: (0, i)),
        out_shape=jax.ShapeDtypeStruct((B, V), jnp.float32),
        compiler_params=pltpu.CompilerParams(
            dimension_semantics=("arbitrary",),
        ),
    )


def kernel(context_ids, emb_table, proj_weight):
    B, CTX = context_ids.shape
    V, D = emb_table.shape
    ids = context_ids.reshape(-1).astype(jnp.int32)
    pooled = _make_gather_pool(B, CTX, D)(ids, emb_table)
    return _make_project(B, D, V, 2048)(pooled, proj_weight)
